# Initial kernel scaffold; baseline (speedup 1.0000x reference)
import jax
import jax.numpy as jnp
from jax.experimental import pallas as pl


def kernel(fatoms, fbonds, agraph, bgraph, scope, W_i, W_h, W_o_w, W_o_b):
    raise NotImplementedError("write your pallas kernel here")


# speedup vs baseline: 7.7308x; 7.7308x over previous
import functools

import jax
import jax.numpy as jnp
from jax import lax
from jax.experimental import pallas as pl
from jax.experimental.pallas import tpu as pltpu
from jax.experimental.pallas import tpu_sc as plsc

ATOM_FDIM = 39
BOND_FDIM = 11
HIDDEN = 128
DEPTH = 6
N_ATOMS = 50000
N_BONDS = 100000
MAX_NB = 6
N_MOLS = 2000
MOL_LEN = 25

NUM_WORKERS = 32
SC_CHUNK = 512
N_BONDS_PAD = -(-N_BONDS // SC_CHUNK) * SC_CHUNK
N_ATOMS_PAD = -(-N_ATOMS // SC_CHUNK) * SC_CHUNK


def _gather_sum_sc(message, idx_c, n_rows_pad):
    n_chunks = n_rows_pad // SC_CHUNK
    assert n_chunks * SC_CHUNK == n_rows_pad
    per_worker = (n_chunks + NUM_WORKERS - 1) // NUM_WORKERS
    mesh = plsc.VectorSubcoreMesh(core_axis_name="c", subcore_axis_name="s")

    @functools.partial(
        pl.kernel,
        out_type=jax.ShapeDtypeStruct((n_rows_pad, HIDDEN), jnp.float32),
        mesh=mesh,
        scratch_types=[
            pltpu.VMEM((MAX_NB * SC_CHUNK,), jnp.int32),
            pltpu.VMEM((SC_CHUNK, HIDDEN), jnp.float32),
            pltpu.SemaphoreType.DMA,
            pltpu.SemaphoreType.DMA,
        ],
    )
    def k(msg_hbm, idx_hbm, out_hbm, idx_v, acc_v, sem0, sem1):
        wid = lax.axis_index("s") * 2 + lax.axis_index("c")
        for i in range(per_worker):
            cid = wid + i * NUM_WORKERS
            @pl.when(cid < n_chunks)
            def _():
                base = cid * SC_CHUNK
                pltpu.sync_copy(idx_hbm.at[cid], idx_v)
                pltpu.async_copy(
                    msg_hbm.at[idx_v.at[pl.ds(0, SC_CHUNK)]], acc_v,
                    sem0).wait()
                cps = [
                    pltpu.async_copy(
                        msg_hbm.at[idx_v.at[pl.ds(j * SC_CHUNK, SC_CHUNK)]],
                        acc_v, sem1, add=True)
                    for j in range(1, MAX_NB)
                ]
                for cp in cps:
                    cp.wait()
                pltpu.sync_copy(acc_v, out_hbm.at[pl.ds(base, SC_CHUNK)])

    return k(message, idx_c)


def _mm_init(fbonds, W_i):
    B = 2000
    grid = N_BONDS // B

    def body(f_ref, w_ref, bin_ref, msg_ref):
        acc = lax.dot_general(f_ref[...], w_ref[...],
                              (((1,), (1,)), ((), ())),
                              preferred_element_type=jnp.float32)
        bin_ref[...] = acc
        msg_ref[...] = jnp.maximum(acc, 0.0)

    return pl.pallas_call(
        body,
        grid=(grid,),
        in_specs=[
            pl.BlockSpec((B, ATOM_FDIM + BOND_FDIM), lambda i: (i, 0)),
            pl.BlockSpec((HIDDEN, ATOM_FDIM + BOND_FDIM), lambda i: (0, 0)),
        ],
        out_specs=[
            pl.BlockSpec((B, HIDDEN), lambda i: (i, 0)),
            pl.BlockSpec((B, HIDDEN), lambda i: (i, 0)),
        ],
        out_shape=[
            jax.ShapeDtypeStruct((N_BONDS, HIDDEN), jnp.float32),
            jax.ShapeDtypeStruct((N_BONDS, HIDDEN), jnp.float32),
        ],
    )(fbonds, W_i)


def _mm_h(nei, binput, W_h):
    B = 2000
    grid = N_BONDS // B

    def body(n_ref, b_ref, w_ref, o_ref):
        acc = lax.dot_general(n_ref[...], w_ref[...],
                              (((1,), (1,)), ((), ())),
                              preferred_element_type=jnp.float32)
        o_ref[...] = jnp.maximum(b_ref[...] + acc, 0.0)

    return pl.pallas_call(
        body,
        grid=(grid,),
        in_specs=[
            pl.BlockSpec((B, HIDDEN), lambda i: (i, 0)),
            pl.BlockSpec((B, HIDDEN), lambda i: (i, 0)),
            pl.BlockSpec((HIDDEN, HIDDEN), lambda i: (0, 0)),
        ],
        out_specs=pl.BlockSpec((B, HIDDEN), lambda i: (i, 0)),
        out_shape=jax.ShapeDtypeStruct((N_BONDS, HIDDEN), jnp.float32),
    )(nei, binput, W_h)


def _final(fatoms, nei_a, W_oa, W_om, W_o_b):
    B = 2000
    MB = B // MOL_LEN
    grid = N_ATOMS // B

    def body(f_ref, n_ref, wa_ref, wm_ref, b_ref, o_ref):
        h = lax.dot_general(f_ref[...], wa_ref[...],
                            (((1,), (1,)), ((), ())),
                            preferred_element_type=jnp.float32)
        h = h + lax.dot_general(n_ref[...], wm_ref[...],
                                (((1,), (1,)), ((), ())),
                                preferred_element_type=jnp.float32)
        h = jnp.maximum(h + b_ref[...], 0.0)
        mrow = lax.broadcasted_iota(jnp.int32, (MB, B), 0)
        acol = lax.broadcasted_iota(jnp.int32, (MB, B), 1) // MOL_LEN
        P = jnp.where(mrow == acol, 1.0 / MOL_LEN, 0.0).astype(jnp.float32)
        o_ref[...] = lax.dot_general(P, h, (((1,), (0,)), ((), ())),
                                     preferred_element_type=jnp.float32)

    return pl.pallas_call(
        body,
        grid=(grid,),
        in_specs=[
            pl.BlockSpec((B, ATOM_FDIM), lambda i: (i, 0)),
            pl.BlockSpec((B, HIDDEN), lambda i: (i, 0)),
            pl.BlockSpec((HIDDEN, ATOM_FDIM), lambda i: (0, 0)),
            pl.BlockSpec((HIDDEN, HIDDEN), lambda i: (0, 0)),
            pl.BlockSpec((1, HIDDEN), lambda i: (0, 0)),
        ],
        out_specs=pl.BlockSpec((MB, HIDDEN), lambda i: (i, 0)),
        out_shape=jax.ShapeDtypeStruct((N_MOLS, HIDDEN), jnp.float32),
    )(fatoms, nei_a, W_oa, W_om, W_o_b)


def kernel(fatoms, fbonds, agraph, bgraph, scope, W_i, W_h, W_o_w, W_o_b):
    del scope
    bgraph_p = jnp.pad(bgraph, ((0, N_BONDS_PAD - N_BONDS), (0, 0)))
    bgraph_c = bgraph_p.reshape(N_BONDS_PAD // SC_CHUNK, SC_CHUNK, MAX_NB)
    bgraph_c = bgraph_c.transpose(0, 2, 1)
    bgraph_c = bgraph_c.reshape(-1, MAX_NB * SC_CHUNK)
    agraph_p = jnp.pad(agraph, ((0, N_ATOMS_PAD - N_ATOMS), (0, 0)))
    agraph_c = agraph_p.reshape(N_ATOMS_PAD // SC_CHUNK, SC_CHUNK, MAX_NB)
    agraph_c = agraph_c.transpose(0, 2, 1)
    agraph_c = agraph_c.reshape(-1, MAX_NB * SC_CHUNK)
    binput, message = _mm_init(fbonds, W_i)
    for _ in range(DEPTH - 1):
        nei = _gather_sum_sc(message, bgraph_c, N_BONDS_PAD)
        message = _mm_h(nei, binput, W_h)
    nei_a = _gather_sum_sc(message, agraph_c, N_ATOMS_PAD)
    W_oa = W_o_w[:, :ATOM_FDIM]
    W_om = W_o_w[:, ATOM_FDIM:]
    return _final(fatoms, nei_a, W_oa, W_om, W_o_b.reshape(1, HIDDEN))


# speedup vs baseline: 11.0660x; 1.4314x over previous
import functools

import jax
import jax.numpy as jnp
from jax import lax
from jax.experimental import pallas as pl
from jax.experimental.pallas import tpu as pltpu
from jax.experimental.pallas import tpu_sc as plsc

ATOM_FDIM = 39
BOND_FDIM = 11
HIDDEN = 128
DEPTH = 6
N_ATOMS = 50000
N_BONDS = 100000
MAX_NB = 6
N_MOLS = 2000
MOL_LEN = 25

NUM_WORKERS = 32
SC_CHUNK = 256
NIDX = MAX_NB * SC_CHUNK
N_BONDS_PAD = -(-N_BONDS // SC_CHUNK) * SC_CHUNK
N_ATOMS_PAD = -(-N_ATOMS // SC_CHUNK) * SC_CHUNK


def _gather_sum_sc(message, idx_c, n_rows_pad):
    n_chunks = n_rows_pad // SC_CHUNK
    assert n_chunks * SC_CHUNK == n_rows_pad
    per_worker = (n_chunks + NUM_WORKERS - 1) // NUM_WORKERS
    mesh = plsc.VectorSubcoreMesh(core_axis_name="c", subcore_axis_name="s")

    @functools.partial(
        pl.kernel,
        out_type=jax.ShapeDtypeStruct((n_rows_pad, HIDDEN), jnp.float32),
        mesh=mesh,
        scratch_types=[
            pltpu.VMEM((per_worker * NIDX,), jnp.int32),
            pltpu.VMEM((SC_CHUNK, HIDDEN), jnp.float32),
            pltpu.VMEM((SC_CHUNK, HIDDEN), jnp.float32),
            pltpu.SemaphoreType.DMA,
            [pltpu.SemaphoreType.DMA] * 2,
            [pltpu.SemaphoreType.DMA] * 2,
            [pltpu.SemaphoreType.DMA] * 2,
        ],
    )
    def k(msg_hbm, idx_hbm, out_hbm, idx_all, acc_a, acc_b,
          semx, semi, sema, sems):
        wid = lax.axis_index("s") * 2 + lax.axis_index("c")
        accs = [acc_a, acc_b]

        def cid(k_):
            return wid + k_ * NUM_WORKERS

        def valid(k_):
            return cid(k_) < n_chunks

        def idx_slice(k_, j):
            return idx_all.at[pl.ds(k_ * NIDX + j * SC_CHUNK, SC_CHUNK)]

        def init_gather(k_):
            return pltpu.async_copy(
                msg_hbm.at[idx_slice(k_, 0)], accs[k_ % 2], semi[k_ % 2])

        def wait_init(k_):
            pltpu.make_async_copy(
                msg_hbm.at[idx_slice(k_, 0)], accs[k_ % 2],
                semi[k_ % 2]).wait()

        def store(k_):
            return pltpu.async_copy(
                accs[k_ % 2],
                out_hbm.at[pl.ds(cid(k_) * SC_CHUNK, SC_CHUNK)],
                sems[k_ % 2])

        def wait_store(k_):
            pltpu.make_async_copy(
                accs[k_ % 2],
                out_hbm.at[pl.ds(cid(k_) * SC_CHUNK, SC_CHUNK)],
                sems[k_ % 2]).wait()

        for k_ in range(per_worker):
            @pl.when(valid(k_))
            def _(k_=k_):
                pltpu.async_copy(
                    idx_hbm.at[cid(k_)],
                    idx_all.at[pl.ds(k_ * NIDX, NIDX)], semx)
        for k_ in range(per_worker):
            @pl.when(valid(k_))
            def _(k_=k_):
                pltpu.make_async_copy(
                    idx_hbm.at[cid(k_)],
                    idx_all.at[pl.ds(k_ * NIDX, NIDX)], semx).wait()

        @pl.when(valid(0))
        def _():
            init_gather(0)

        for k_ in range(per_worker):
            p = k_ % 2

            @pl.when(valid(k_))
            def _(k_=k_, p=p):
                wait_init(k_)
                for j in range(1, MAX_NB):
                    pltpu.async_copy(
                        msg_hbm.at[idx_slice(k_, j)], accs[p], sema[p],
                        add=True)

            if k_ + 1 < per_worker:
                @pl.when(valid(k_ + 1))
                def _(k_=k_):
                    if k_ >= 1:
                        wait_store(k_ - 1)
                    init_gather(k_ + 1)

            @pl.when(valid(k_))
            def _(k_=k_, p=p):
                for j in range(1, MAX_NB):
                    pltpu.make_async_copy(
                        msg_hbm.at[idx_slice(k_, j)], accs[p],
                        sema[p]).wait()
                store(k_)

        for k_ in range(per_worker):
            @pl.when(valid(k_) & (cid(k_) + 2 * NUM_WORKERS >= n_chunks))
            def _(k_=k_):
                wait_store(k_)

    return k(message, idx_c)


def _mm_init(fbonds, W_i):
    B = 2000
    grid = N_BONDS // B

    def body(f_ref, w_ref, bin_ref, msg_ref):
        acc = lax.dot_general(f_ref[...], w_ref[...],
                              (((1,), (1,)), ((), ())),
                              preferred_element_type=jnp.float32)
        bin_ref[...] = acc
        msg_ref[...] = jnp.maximum(acc, 0.0)

    return pl.pallas_call(
        body,
        grid=(grid,),
        in_specs=[
            pl.BlockSpec((B, ATOM_FDIM + BOND_FDIM), lambda i: (i, 0)),
            pl.BlockSpec((HIDDEN, ATOM_FDIM + BOND_FDIM), lambda i: (0, 0)),
        ],
        out_specs=[
            pl.BlockSpec((B, HIDDEN), lambda i: (i, 0)),
            pl.BlockSpec((B, HIDDEN), lambda i: (i, 0)),
        ],
        out_shape=[
            jax.ShapeDtypeStruct((N_BONDS, HIDDEN), jnp.float32),
            jax.ShapeDtypeStruct((N_BONDS, HIDDEN), jnp.float32),
        ],
    )(fbonds, W_i)


def _mm_h(nei, binput, W_h):
    B = 2000
    grid = N_BONDS // B

    def body(n_ref, b_ref, w_ref, o_ref):
        acc = lax.dot_general(n_ref[...], w_ref[...],
                              (((1,), (1,)), ((), ())),
                              preferred_element_type=jnp.float32)
        o_ref[...] = jnp.maximum(b_ref[...] + acc, 0.0)

    return pl.pallas_call(
        body,
        grid=(grid,),
        in_specs=[
            pl.BlockSpec((B, HIDDEN), lambda i: (i, 0)),
            pl.BlockSpec((B, HIDDEN), lambda i: (i, 0)),
            pl.BlockSpec((HIDDEN, HIDDEN), lambda i: (0, 0)),
        ],
        out_specs=pl.BlockSpec((B, HIDDEN), lambda i: (i, 0)),
        out_shape=jax.ShapeDtypeStruct((N_BONDS, HIDDEN), jnp.float32),
    )(nei, binput, W_h)


def _final(fatoms, nei_a, W_oa, W_om, W_o_b):
    B = 2000
    MB = B // MOL_LEN
    grid = N_ATOMS // B

    def body(f_ref, n_ref, wa_ref, wm_ref, b_ref, o_ref):
        h = lax.dot_general(f_ref[...], wa_ref[...],
                            (((1,), (1,)), ((), ())),
                            preferred_element_type=jnp.float32)
        h = h + lax.dot_general(n_ref[...], wm_ref[...],
                                (((1,), (1,)), ((), ())),
                                preferred_element_type=jnp.float32)
        h = jnp.maximum(h + b_ref[...], 0.0)
        mrow = lax.broadcasted_iota(jnp.int32, (MB, B), 0)
        acol = lax.broadcasted_iota(jnp.int32, (MB, B), 1) // MOL_LEN
        P = jnp.where(mrow == acol, 1.0 / MOL_LEN, 0.0).astype(jnp.float32)
        o_ref[...] = lax.dot_general(P, h, (((1,), (0,)), ((), ())),
                                     preferred_element_type=jnp.float32)

    return pl.pallas_call(
        body,
        grid=(grid,),
        in_specs=[
            pl.BlockSpec((B, ATOM_FDIM), lambda i: (i, 0)),
            pl.BlockSpec((B, HIDDEN), lambda i: (i, 0)),
            pl.BlockSpec((HIDDEN, ATOM_FDIM), lambda i: (0, 0)),
            pl.BlockSpec((HIDDEN, HIDDEN), lambda i: (0, 0)),
            pl.BlockSpec((1, HIDDEN), lambda i: (0, 0)),
        ],
        out_specs=pl.BlockSpec((MB, HIDDEN), lambda i: (i, 0)),
        out_shape=jax.ShapeDtypeStruct((N_MOLS, HIDDEN), jnp.float32),
    )(fatoms, nei_a, W_oa, W_om, W_o_b)


def kernel(fatoms, fbonds, agraph, bgraph, scope, W_i, W_h, W_o_w, W_o_b):
    del scope
    bgraph_p = jnp.pad(bgraph, ((0, N_BONDS_PAD - N_BONDS), (0, 0)))
    bgraph_c = bgraph_p.reshape(N_BONDS_PAD // SC_CHUNK, SC_CHUNK, MAX_NB)
    bgraph_c = bgraph_c.transpose(0, 2, 1)
    bgraph_c = bgraph_c.reshape(-1, MAX_NB * SC_CHUNK)
    agraph_p = jnp.pad(agraph, ((0, N_ATOMS_PAD - N_ATOMS), (0, 0)))
    agraph_c = agraph_p.reshape(N_ATOMS_PAD // SC_CHUNK, SC_CHUNK, MAX_NB)
    agraph_c = agraph_c.transpose(0, 2, 1)
    agraph_c = agraph_c.reshape(-1, MAX_NB * SC_CHUNK)
    binput, message = _mm_init(fbonds, W_i)
    for _ in range(DEPTH - 1):
        nei = _gather_sum_sc(message, bgraph_c, N_BONDS_PAD)
        message = _mm_h(nei, binput, W_h)
    nei_a = _gather_sum_sc(message, agraph_c, N_ATOMS_PAD)
    W_oa = W_o_w[:, :ATOM_FDIM]
    W_om = W_o_w[:, ATOM_FDIM:]
    return _final(fatoms, nei_a, W_oa, W_om, W_o_b.reshape(1, HIDDEN))
